# initial kernel scaffold (unmeasured)
import jax
import jax.numpy as jnp
from jax import lax
from jax.experimental import pallas as pl
from jax.experimental.pallas import tpu as pltpu

N_DEV = 8
M = 2048
D = 2048
R = M // N_DEV


def _body(
    x_ref,
    resid_ref,
    gamma_ref,
    out_ref,
    rs_send,
    rs_recv,
    rs_send_sem,
    rs_recv_sem,
    ag_send_sem,
    ag_recv_sem,
):
    i = lax.axis_index("i")
    right = lax.rem(i + 1, N_DEV)
    left = lax.rem(i + N_DEV - 1, N_DEV)

    barrier_sem = pltpu.get_barrier_semaphore()
    for nbr in (left, right):
        pl.semaphore_signal(
            barrier_sem,
            inc=1,
            device_id=(nbr,),
            device_id_type=pl.DeviceIdType.MESH,
        )
    pl.semaphore_wait(barrier_sem, 2)

    for s in range(N_DEV - 1):
        slot = s % 2
        if s == 0:
            c = lax.rem(i, N_DEV)
            rs_send[slot, :, :] = x_ref[pl.ds(c * R, R), :].astype(jnp.bfloat16)
        else:
            c = lax.rem(i - s + N_DEV, N_DEV)
            acc = rs_recv[s - 1, :, :].astype(jnp.float32) + x_ref[pl.ds(c * R, R), :]
            rs_send[slot, :, :] = acc.astype(jnp.bfloat16)
        rdma = pltpu.make_async_remote_copy(
            src_ref=rs_send.at[slot],
            dst_ref=rs_recv.at[s],
            send_sem=rs_send_sem.at[slot],
            recv_sem=rs_recv_sem.at[s],
            device_id=(right,),
            device_id_type=pl.DeviceIdType.MESH,
        )
        rdma.start()
        rdma.wait()

    cstar = lax.rem(i + 1, N_DEV)
    rows = pl.ds(cstar * R, R)
    y = (
        rs_recv[N_DEV - 2, :, :].astype(jnp.float32)
        + x_ref[rows, :]
        + resid_ref[rows, :]
    )
    rms = jnp.sqrt(jnp.mean(y * y, axis=-1, keepdims=True) + 1e-6)
    out_ref[rows, :] = (y / rms) * gamma_ref[:, :]

    for s in range(N_DEV - 1):
        c = lax.rem(i + 1 - s + N_DEV, N_DEV)
        rows_c = pl.ds(c * R, R)
        rdma = pltpu.make_async_remote_copy(
            src_ref=out_ref.at[rows_c],
            dst_ref=out_ref.at[rows_c],
            send_sem=ag_send_sem.at[s % 2],
            recv_sem=ag_recv_sem.at[s],
            device_id=(right,),
            device_id_type=pl.DeviceIdType.MESH,
        )
        rdma.start()
        rdma.wait()


def kernel(partial, resid, gamma):
    x = partial.reshape(M, D)
    g = gamma.reshape(1, D)
    return pl.pallas_call(
        _body,
        out_shape=jax.ShapeDtypeStruct((M, D), jnp.float32),
        in_specs=[
            pl.BlockSpec(memory_space=pltpu.VMEM),
            pl.BlockSpec(memory_space=pltpu.VMEM),
            pl.BlockSpec(memory_space=pltpu.VMEM),
        ],
        out_specs=pl.BlockSpec(memory_space=pltpu.VMEM),
        scratch_shapes=[
            pltpu.VMEM((2, R, D), jnp.bfloat16),
            pltpu.VMEM((N_DEV - 1, R, D), jnp.bfloat16),
            pltpu.SemaphoreType.DMA((2,)),
            pltpu.SemaphoreType.DMA((N_DEV - 1,)),
            pltpu.SemaphoreType.DMA((2,)),
            pltpu.SemaphoreType.DMA((N_DEV - 1,)),
        ],
        compiler_params=pltpu.CompilerParams(collective_id=0),
    )(x, resid, g)


# baseline (device time: 295383 ns/iter reference)
import jax
import jax.numpy as jnp
from jax import lax
from jax.experimental import pallas as pl
from jax.experimental.pallas import tpu as pltpu

N_DEV = 8
M = 2048
D = 2048
R = M // N_DEV


def _body(
    x_ref,
    resid_ref,
    gamma_ref,
    out_ref,
    rs_send,
    rs_recv,
    rs_send_sem,
    rs_recv_sem,
    ag_send_sem,
    ag_recv_sem,
):
    i = lax.axis_index("i")
    right = lax.rem(i + 1, N_DEV)
    left = lax.rem(i + N_DEV - 1, N_DEV)

    barrier_sem = pltpu.get_barrier_semaphore()
    for nbr in (left, right):
        pl.semaphore_signal(
            barrier_sem,
            inc=1,
            device_id=(nbr,),
            device_id_type=pl.DeviceIdType.MESH,
        )
    pl.semaphore_wait(barrier_sem, 2)

    for s in range(N_DEV - 1):
        slot = s % 2
        if s == 0:
            c = lax.rem(i, N_DEV)
            rs_send[slot, :, :] = x_ref[pl.ds(c * R, R), :].astype(jnp.bfloat16)
        else:
            c = lax.rem(i - s + N_DEV, N_DEV)
            acc = rs_recv[s - 1, :, :].astype(jnp.float32) + x_ref[pl.ds(c * R, R), :]
            rs_send[slot, :, :] = acc.astype(jnp.bfloat16)
        rdma = pltpu.make_async_remote_copy(
            src_ref=rs_send.at[slot],
            dst_ref=rs_recv.at[s],
            send_sem=rs_send_sem.at[slot],
            recv_sem=rs_recv_sem.at[s],
            device_id=(right,),
            device_id_type=pl.DeviceIdType.MESH,
        )
        rdma.start()
        rdma.wait()

    cstar = lax.rem(i + 1, N_DEV)
    rows = pl.ds(cstar * R, R)
    y = (
        rs_recv[N_DEV - 2, :, :].astype(jnp.float32)
        + x_ref[rows, :]
        + resid_ref[rows, :]
    )
    rms = jnp.sqrt(jnp.mean(y * y, axis=-1, keepdims=True) + 1e-6)
    out_ref[rows, :] = (y / rms) * gamma_ref[:, :]

    for s in range(N_DEV - 1):
        c = lax.rem(i + 1 - s + N_DEV, N_DEV)
        rows_c = pl.ds(c * R, R)
        rdma = pltpu.make_async_remote_copy(
            src_ref=out_ref.at[rows_c],
            dst_ref=out_ref.at[rows_c],
            send_sem=ag_send_sem.at[s % 2],
            recv_sem=ag_recv_sem.at[s],
            device_id=(right,),
            device_id_type=pl.DeviceIdType.MESH,
        )
        rdma.start()
        rdma.wait()


def kernel(partial, resid, gamma):
    x = partial.reshape(M, D)
    g = gamma.reshape(1, D)
    return pl.pallas_call(
        _body,
        out_shape=jax.ShapeDtypeStruct((M, D), jnp.float32),
        in_specs=[
            pl.BlockSpec(memory_space=pltpu.VMEM),
            pl.BlockSpec(memory_space=pltpu.VMEM),
            pl.BlockSpec(memory_space=pltpu.VMEM),
        ],
        out_specs=pl.BlockSpec(memory_space=pltpu.VMEM),
        scratch_shapes=[
            pltpu.VMEM((2, R, D), jnp.bfloat16),
            pltpu.VMEM((N_DEV - 1, R, D), jnp.bfloat16),
            pltpu.SemaphoreType.DMA((2,)),
            pltpu.SemaphoreType.DMA((N_DEV - 1,)),
            pltpu.SemaphoreType.DMA((2,)),
            pltpu.SemaphoreType.DMA((N_DEV - 1,)),
        ],
        compiler_params=pltpu.CompilerParams(
            collective_id=0, vmem_limit_bytes=96 * 1024 * 1024
        ),
    )(x, resid, g)


# device time: 136087 ns/iter; 2.1705x vs baseline; 2.1705x over previous
import jax
import jax.numpy as jnp
from jax import lax
from jax.experimental import pallas as pl
from jax.experimental.pallas import tpu as pltpu

N_DEV = 8
M = 2048
D = 2048
H = M // 2
R = H // N_DEV


def _rows0(c):
    return pl.ds(c * R, R)


def _rows1(c):
    return pl.ds(H + c * R, R)


def _body(
    x_ref,
    resid_ref,
    gamma_ref,
    out_ref,
    rs_send_r,
    rs_recv_r,
    rs_send_l,
    rs_recv_l,
    rs_send_sem_r,
    rs_recv_sem_r,
    rs_send_sem_l,
    rs_recv_sem_l,
    ag_send_sem_r,
    ag_recv_sem_r,
    ag_send_sem_l,
    ag_recv_sem_l,
):
    i = lax.axis_index("i")
    right = lax.rem(i + 1, N_DEV)
    left = lax.rem(i + N_DEV - 1, N_DEV)

    barrier_sem = pltpu.get_barrier_semaphore()
    for nbr in (left, right):
        pl.semaphore_signal(
            barrier_sem,
            inc=1,
            device_id=(nbr,),
            device_id_type=pl.DeviceIdType.MESH,
        )
    pl.semaphore_wait(barrier_sem, 2)

    for s in range(N_DEV - 1):
        slot = s % 2
        if s == 0:
            rs_send_r[slot, :, :] = x_ref[_rows0(i), :].astype(jnp.bfloat16)
            rs_send_l[slot, :, :] = x_ref[_rows1(i), :].astype(jnp.bfloat16)
        else:
            cr = lax.rem(i - s + N_DEV, N_DEV)
            cl = lax.rem(i + s, N_DEV)
            acc_r = rs_recv_r[s - 1, :, :].astype(jnp.float32) + x_ref[_rows0(cr), :]
            acc_l = rs_recv_l[s - 1, :, :].astype(jnp.float32) + x_ref[_rows1(cl), :]
            rs_send_r[slot, :, :] = acc_r.astype(jnp.bfloat16)
            rs_send_l[slot, :, :] = acc_l.astype(jnp.bfloat16)
        rdma_r = pltpu.make_async_remote_copy(
            src_ref=rs_send_r.at[slot],
            dst_ref=rs_recv_r.at[s],
            send_sem=rs_send_sem_r.at[slot],
            recv_sem=rs_recv_sem_r.at[s],
            device_id=(right,),
            device_id_type=pl.DeviceIdType.MESH,
        )
        rdma_l = pltpu.make_async_remote_copy(
            src_ref=rs_send_l.at[slot],
            dst_ref=rs_recv_l.at[s],
            send_sem=rs_send_sem_l.at[slot],
            recv_sem=rs_recv_sem_l.at[s],
            device_id=(left,),
            device_id_type=pl.DeviceIdType.MESH,
        )
        rdma_r.start()
        rdma_l.start()
        rdma_r.wait()
        rdma_l.wait()

    g = gamma_ref[:, :]
    c0 = lax.rem(i + 1, N_DEV)
    y0 = (
        rs_recv_r[N_DEV - 2, :, :].astype(jnp.float32)
        + x_ref[_rows0(c0), :]
        + resid_ref[_rows0(c0), :]
    )
    rms0 = jnp.sqrt(jnp.mean(y0 * y0, axis=-1, keepdims=True) + 1e-6)
    out_ref[_rows0(c0), :] = ((y0 / rms0) * g).astype(jnp.bfloat16)

    c1 = lax.rem(i + N_DEV - 1, N_DEV)
    y1 = (
        rs_recv_l[N_DEV - 2, :, :].astype(jnp.float32)
        + x_ref[_rows1(c1), :]
        + resid_ref[_rows1(c1), :]
    )
    rms1 = jnp.sqrt(jnp.mean(y1 * y1, axis=-1, keepdims=True) + 1e-6)
    out_ref[_rows1(c1), :] = ((y1 / rms1) * g).astype(jnp.bfloat16)

    for s in range(N_DEV - 1):
        cr = lax.rem(i + 1 - s + N_DEV, N_DEV)
        cl = lax.rem(i + N_DEV - 1 + s, N_DEV)
        rdma_r = pltpu.make_async_remote_copy(
            src_ref=out_ref.at[_rows0(cr)],
            dst_ref=out_ref.at[_rows0(cr)],
            send_sem=ag_send_sem_r.at[s % 2],
            recv_sem=ag_recv_sem_r.at[s],
            device_id=(right,),
            device_id_type=pl.DeviceIdType.MESH,
        )
        rdma_l = pltpu.make_async_remote_copy(
            src_ref=out_ref.at[_rows1(cl)],
            dst_ref=out_ref.at[_rows1(cl)],
            send_sem=ag_send_sem_l.at[s % 2],
            recv_sem=ag_recv_sem_l.at[s],
            device_id=(left,),
            device_id_type=pl.DeviceIdType.MESH,
        )
        rdma_r.start()
        rdma_l.start()
        rdma_r.wait()
        rdma_l.wait()


def kernel(partial, resid, gamma):
    x = partial.reshape(M, D)
    g = gamma.reshape(1, D)
    return pl.pallas_call(
        _body,
        out_shape=jax.ShapeDtypeStruct((M, D), jnp.bfloat16),
        in_specs=[
            pl.BlockSpec(memory_space=pltpu.VMEM),
            pl.BlockSpec(memory_space=pltpu.VMEM),
            pl.BlockSpec(memory_space=pltpu.VMEM),
        ],
        out_specs=pl.BlockSpec(memory_space=pltpu.VMEM),
        scratch_shapes=[
            pltpu.VMEM((2, R, D), jnp.bfloat16),
            pltpu.VMEM((N_DEV - 1, R, D), jnp.bfloat16),
            pltpu.VMEM((2, R, D), jnp.bfloat16),
            pltpu.VMEM((N_DEV - 1, R, D), jnp.bfloat16),
            pltpu.SemaphoreType.DMA((2,)),
            pltpu.SemaphoreType.DMA((N_DEV - 1,)),
            pltpu.SemaphoreType.DMA((2,)),
            pltpu.SemaphoreType.DMA((N_DEV - 1,)),
            pltpu.SemaphoreType.DMA((2,)),
            pltpu.SemaphoreType.DMA((N_DEV - 1,)),
            pltpu.SemaphoreType.DMA((2,)),
            pltpu.SemaphoreType.DMA((N_DEV - 1,)),
        ],
        compiler_params=pltpu.CompilerParams(
            collective_id=0, vmem_limit_bytes=96 * 1024 * 1024
        ),
    )(x, resid, g)


# device time: 106860 ns/iter; 2.7642x vs baseline; 1.2735x over previous
import jax
import jax.numpy as jnp
from jax import lax
from jax.experimental import pallas as pl
from jax.experimental.pallas import tpu as pltpu

N_DEV = 8
M = 2048
D = 2048
H = M // 2
R = H // N_DEV
NSUB = 2
SUB = R // NSUB
NH = N_DEV - 1


def _body(
    x_ref,
    resid_ref,
    gamma_ref,
    out_ref,
    rs_send_r,
    rs_recv_r,
    rs_send_l,
    rs_recv_l,
    rs_send_sem_r,
    rs_recv_sem_r,
    rs_send_sem_l,
    rs_recv_sem_l,
    ag_send_sem_r,
    ag_recv_sem_r,
    ag_send_sem_l,
    ag_recv_sem_l,
):
    i = lax.axis_index("i")
    right = lax.rem(i + 1, N_DEV)
    left = lax.rem(i + N_DEV - 1, N_DEV)

    dirs = {
        "r": dict(
            dev=right,
            base=0,
            rs_send=rs_send_r,
            rs_recv=rs_recv_r,
            rs_ssem=rs_send_sem_r,
            rs_rsem=rs_recv_sem_r,
            ag_ssem=ag_send_sem_r,
            ag_rsem=ag_recv_sem_r,
            rs_chunk=lambda s: lax.rem(i - s + N_DEV, N_DEV),
            ag_chunk=lambda s: lax.rem(i + 1 - s + N_DEV, N_DEV),
        ),
        "l": dict(
            dev=left,
            base=H,
            rs_send=rs_send_l,
            rs_recv=rs_recv_l,
            rs_ssem=rs_send_sem_l,
            rs_rsem=rs_recv_sem_l,
            ag_ssem=ag_send_sem_l,
            ag_rsem=ag_recv_sem_l,
            rs_chunk=lambda s: lax.rem(i + s, N_DEV),
            ag_chunk=lambda s: lax.rem(i + N_DEV - 1 + s, N_DEV),
        ),
    }

    def sub_rows(base, c, j):
        return pl.ds(base + c * R + j * SUB, SUB)

    barrier_sem = pltpu.get_barrier_semaphore()
    for nbr in (left, right):
        pl.semaphore_signal(
            barrier_sem,
            inc=1,
            device_id=(nbr,),
            device_id_type=pl.DeviceIdType.MESH,
        )
    pl.semaphore_wait(barrier_sem, 2)

    rs_desc = {}

    def rs_start(s, d, j):
        dd = dirs[d]
        desc = pltpu.make_async_remote_copy(
            src_ref=dd["rs_send"].at[s % 2, pl.ds(j * SUB, SUB)],
            dst_ref=dd["rs_recv"].at[s, pl.ds(j * SUB, SUB)],
            send_sem=dd["rs_ssem"].at[s % 2, j],
            recv_sem=dd["rs_rsem"].at[s, j],
            device_id=(dd["dev"],),
            device_id_type=pl.DeviceIdType.MESH,
        )
        rs_desc[(s, d, j)] = desc
        desc.start()

    for d in ("r", "l"):
        dd = dirs[d]
        dd["rs_send"][0, :, :] = x_ref[
            pl.ds(dd["base"] + i * R, R), :
        ].astype(jnp.bfloat16)
    for j in range(NSUB):
        for d in ("r", "l"):
            rs_start(0, d, j)

    for s in range(1, NH):
        for j in range(NSUB):
            for d in ("r", "l"):
                dd = dirs[d]
                rs_desc[(s - 1, d, j)].wait_recv()
                if s >= 2:
                    rs_desc[(s - 2, d, j)].wait_send()
                c = dd["rs_chunk"](s)
                acc = (
                    dd["rs_recv"][s - 1, pl.ds(j * SUB, SUB), :].astype(jnp.float32)
                    + x_ref[sub_rows(dd["base"], c, j), :]
                )
                dd["rs_send"][s % 2, pl.ds(j * SUB, SUB), :] = acc.astype(
                    jnp.bfloat16
                )
                rs_start(s, d, j)

    g = gamma_ref[:, :]
    ag_desc = {}

    def ag_start(s, d, j):
        dd = dirs[d]
        rows = sub_rows(dd["base"], dd["ag_chunk"](s), j)
        desc = pltpu.make_async_remote_copy(
            src_ref=out_ref.at[rows],
            dst_ref=out_ref.at[rows],
            send_sem=dd["ag_ssem"].at[s % 2, j],
            recv_sem=dd["ag_rsem"].at[s, j],
            device_id=(dd["dev"],),
            device_id_type=pl.DeviceIdType.MESH,
        )
        ag_desc[(s, d, j)] = desc
        desc.start()

    for j in range(NSUB):
        for d in ("r", "l"):
            dd = dirs[d]
            rs_desc[(NH - 1, d, j)].wait_recv()
            c = dd["ag_chunk"](0)
            rows = sub_rows(dd["base"], c, j)
            y = (
                dd["rs_recv"][NH - 1, pl.ds(j * SUB, SUB), :].astype(jnp.float32)
                + x_ref[rows, :]
                + resid_ref[rows, :]
            )
            rms = jnp.sqrt(jnp.mean(y * y, axis=-1, keepdims=True) + 1e-6)
            out_ref[rows, :] = ((y / rms) * g).astype(jnp.bfloat16)
            ag_start(0, d, j)

    for s in range(1, NH):
        for j in range(NSUB):
            for d in ("r", "l"):
                ag_desc[(s - 1, d, j)].wait_recv()
                if s >= 2:
                    ag_desc[(s - 2, d, j)].wait_send()
                ag_start(s, d, j)

    for j in range(NSUB):
        for d in ("r", "l"):
            ag_desc[(NH - 1, d, j)].wait_recv()
            rs_desc[(NH - 2, d, j)].wait_send()
            rs_desc[(NH - 1, d, j)].wait_send()
            ag_desc[(NH - 2, d, j)].wait_send()
            ag_desc[(NH - 1, d, j)].wait_send()


def kernel(partial, resid, gamma):
    x = partial.reshape(M, D)
    g = gamma.reshape(1, D)
    return pl.pallas_call(
        _body,
        out_shape=jax.ShapeDtypeStruct((M, D), jnp.bfloat16),
        in_specs=[
            pl.BlockSpec(memory_space=pltpu.VMEM),
            pl.BlockSpec(memory_space=pltpu.VMEM),
            pl.BlockSpec(memory_space=pltpu.VMEM),
        ],
        out_specs=pl.BlockSpec(memory_space=pltpu.VMEM),
        scratch_shapes=[
            pltpu.VMEM((2, R, D), jnp.bfloat16),
            pltpu.VMEM((NH, R, D), jnp.bfloat16),
            pltpu.VMEM((2, R, D), jnp.bfloat16),
            pltpu.VMEM((NH, R, D), jnp.bfloat16),
            pltpu.SemaphoreType.DMA((2, NSUB)),
            pltpu.SemaphoreType.DMA((NH, NSUB)),
            pltpu.SemaphoreType.DMA((2, NSUB)),
            pltpu.SemaphoreType.DMA((NH, NSUB)),
            pltpu.SemaphoreType.DMA((2, NSUB)),
            pltpu.SemaphoreType.DMA((NH, NSUB)),
            pltpu.SemaphoreType.DMA((2, NSUB)),
            pltpu.SemaphoreType.DMA((NH, NSUB)),
        ],
        compiler_params=pltpu.CompilerParams(
            collective_id=0, vmem_limit_bytes=96 * 1024 * 1024
        ),
    )(x, resid, g)
